# baseline (speedup 1.0000x reference)
import jax
import jax.numpy as jnp
from jax import lax
from jax.experimental import pallas as pl
from jax.experimental.pallas import tpu as pltpu
from jax.experimental.pallas import tpu_sc as plsc

_NUM_SAMPLES = 50000
_NUM_FEATURES = 256
_BATCH = 256
_NUM_CLASSES = 1000
_TEMP = 0.05

_NC = 2
_NS = 16
_NRG = 4
_NCG = 4
_CPAD = 1024
_DUMP = 1000
_K = 320
_NCHUNK = 20
_CW = _NUM_FEATURES // _NCG


def _sc_body(features, labels, indexes, csum_out, cnt_out, tgt_out,
             acc, cacc, fbuf, lbl2, idxbuf, tgtbuf, fsem, lsem, sem):
  cid = lax.axis_index("c")
  sid = lax.axis_index("s")
  wid = sid * _NC + cid
  rl = sid // _NCG
  cg = sid % _NCG
  c0 = cg * _CW

  rg = cid * _NRG + rl
  nrow16 = _NUM_SAMPLES // 16
  lo16 = (rg * nrow16) // (_NC * _NRG)
  hi16 = ((rg + 1) * nrow16) // (_NC * _NRG)
  nq = hi16 - lo16

  def _zero(i, carry):
    for j in range(_CW // 16):
      acc[i, pl.ds(j * 16, 16)] = jnp.zeros((16,), jnp.float32)
    cacc[i, pl.ds(0, 16)] = jnp.zeros((16,), jnp.float32)
    return carry
  lax.fori_loop(0, _CPAD, _zero, 0)

  @pl.when(wid == 0)
  def _targets():
    for off in (0, 128):
      pltpu.sync_copy(indexes.at[pl.ds(off, 128)], idxbuf)
      pltpu.async_copy(labels.at[idxbuf], tgtbuf, sem).wait()
      pltpu.sync_copy(tgtbuf, tgt_out.at[pl.ds(off, 128)])

  ones16 = jnp.ones((16,), jnp.float32)
  iota16 = lax.broadcasted_iota(jnp.int32, (16,), 0)
  is_cnt = cg == 0
  start = lo16 * 16
  end = hi16 * 16

  def _base(c):
    return jnp.minimum(start + c * _K, end - _K)

  def _start_dmas(c, slot):
    b = _base(c)
    fd, ld = fsem.at[slot], lsem.at[slot]
    fdesc = pltpu.make_async_copy(
        features.at[pl.ds(b, _K), pl.ds(c0, _CW)], fbuf.at[slot], fd)
    ldesc = pltpu.make_async_copy(labels.at[pl.ds(b, _K)], lbl2.at[slot], ld)
    fdesc.start()
    ldesc.start()

  def _wait_dmas(c, slot):
    b = _base(c)
    pltpu.make_async_copy(
        features.at[pl.ds(b, _K), pl.ds(c0, _CW)], fbuf.at[slot],
        fsem.at[slot]).wait()
    pltpu.make_async_copy(labels.at[pl.ds(b, _K)], lbl2.at[slot],
                          lsem.at[slot]).wait()

  def _process(c, slot):
    base = _base(c)
    cutoff = start + c * _K

    slot16 = jnp.full((16,), slot, jnp.int32)

    def _rows16(q, carry):
      lv = lbl2[slot, pl.ds(q * 16, 16)]
      rows = base + q * 16 + iota16
      lbl16 = jnp.where(rows >= cutoff, lv, _DUMP)
      row16 = q * 16 + iota16
      for c in range(_CW):
        c16 = jnp.full((16,), c, jnp.int32)
        v = plsc.load_gather(fbuf, [slot16, row16, c16])
        plsc.addupdate_scatter(acc, [lbl16, c16], v)
      @pl.when(is_cnt)
      def _():
        plsc.addupdate_scatter(cacc, [lbl16, jnp.zeros((16,), jnp.int32)],
                               ones16)
      return carry
    lax.fori_loop(0, _K // 16, _rows16, 0)

  _start_dmas(0, 0)
  for c in range(_NCHUNK):
    if c + 1 < _NCHUNK:
      _start_dmas(c + 1, (c + 1) % 2)
    _wait_dmas(c, c % 2)
    _process(c, c % 2)

  pltpu.sync_copy(acc, csum_out.at[cid, sid])
  @pl.when(is_cnt)
  def _dump_cnt():
    pltpu.sync_copy(cacc, cnt_out.at[cid, rl])


@jax.jit
def _sc_stage(features, labels, indexes):
  mesh = plsc.VectorSubcoreMesh(core_axis_name="c", subcore_axis_name="s",
                                num_cores=_NC, num_subcores=_NS)
  return pl.kernel(
      _sc_body,
      out_type=(
          jax.ShapeDtypeStruct((_NC, _NS, _CPAD, _CW), jnp.float32),
          jax.ShapeDtypeStruct((_NC, _NRG, _CPAD, 16), jnp.float32),
          jax.ShapeDtypeStruct((_BATCH,), jnp.int32),
      ),
      mesh=mesh,
      compiler_params=pltpu.CompilerParams(use_tc_tiling_on_sc=False,
                                           needs_layout_passes=False),
      scratch_types=[
          pltpu.VMEM((_CPAD, _CW), jnp.float32),
          pltpu.VMEM((_CPAD, 16), jnp.float32),
          pltpu.VMEM((2, _K, _CW), jnp.float32),
          pltpu.VMEM((2, _K), jnp.int32),
          pltpu.VMEM((128,), jnp.int32),
          pltpu.VMEM((128,), jnp.int32),
          pltpu.SemaphoreType.DMA((2,)),
          pltpu.SemaphoreType.DMA((2,)),
          pltpu.SemaphoreType.DMA,
      ],
  )(features, labels, indexes)


def _tc_body(inp_ref, csum_ref, cnt_ref, tgt_ref, out_ref):
  blocks = []
  for g in range(_NCG):
    blk = csum_ref[0, g]
    for cid in range(_NC):
      for rl in range(_NRG):
        if cid == 0 and rl == 0:
          continue
        blk = blk + csum_ref[cid, rl * _NCG + g]
    blocks.append(blk)
  csum = jnp.concatenate(blocks, axis=1)
  cnt2 = cnt_ref[0, 0]
  for cid in range(_NC):
    for rl in range(_NRG):
      if cid == 0 and rl == 0:
        continue
      cnt2 = cnt2 + cnt_ref[cid, rl]
  sim = lax.dot_general(inp_ref[...], csum, (((1,), (1,)), ((), ())),
                        preferred_element_type=jnp.float32)
  w = jnp.full((1, 16), 1.0, jnp.float32)
  nums = lax.dot_general(w, cnt2, (((1,), (1,)), ((), ())),
                         preferred_element_type=jnp.float32)
  iota_c = lax.broadcasted_iota(jnp.int32, (1, _CPAD), 1)
  maskf = jnp.where(jnp.logical_and(nums > 0.5, iota_c < _NUM_CLASSES),
                    1.0, 0.0).astype(jnp.float32)
  denom = _TEMP * jnp.where(nums > 0.5, nums, 1.0)
  exps = jnp.exp(sim / denom) * maskf
  sums = jnp.sum(exps, axis=1, keepdims=True) + 1e-6
  logp = jnp.log(exps / sums + 1e-6)
  iota_bc = lax.broadcasted_iota(jnp.int32, (_BATCH, _CPAD), 1)
  onehot = jnp.where(iota_bc == tgt_ref[...], 1.0, 0.0)
  picked = jnp.sum(logp * onehot, axis=1, keepdims=True)
  out_ref[...] = -jnp.sum(picked, axis=0, keepdims=True) / float(_BATCH)


@jax.jit
def _tc_stage(inputs, csum_p, cnt_p, tgt):
  return pl.pallas_call(
      _tc_body,
      out_shape=jax.ShapeDtypeStruct((1, 1), jnp.float32),
  )(inputs, csum_p, cnt_p, tgt)


def kernel(inputs, indexes, features, labels):
  csum_p, cnt_p, tgt = _sc_stage(features, labels, indexes)
  loss = _tc_stage(inputs, csum_p, cnt_p, tgt.reshape(_BATCH, 1))
  return loss[0, 0]


# speedup vs baseline: 3.1336x; 3.1336x over previous
import jax
import jax.numpy as jnp
from jax import lax
from jax.experimental import pallas as pl
from jax.experimental.pallas import tpu as pltpu
from jax.experimental.pallas import tpu_sc as plsc

_NUM_SAMPLES = 50000
_NUM_FEATURES = 256
_BATCH = 256
_NUM_CLASSES = 1000
_TEMP = 0.05

_NC = 2
_NS = 16
_NRG = 4
_NCG = 4
_CPAD = 1024
_DUMP = 1000
_K = 320
_NCHUNK = 20
_CW = _NUM_FEATURES // _NCG


def _sc_body(features, labels, indexes, csum_out, cnt_out, tgt_out,
             acc, cacc, fbuf, lbl2, idxbuf, tgtbuf, fsem, lsem, sem):
  cid = lax.axis_index("c")
  sid = lax.axis_index("s")
  wid = sid * _NC + cid
  rl = sid // _NCG
  cg = sid % _NCG
  c0 = cg * _CW

  rg = cid * _NRG + rl
  nrow16 = _NUM_SAMPLES // 16
  lo16 = (rg * nrow16) // (_NC * _NRG)
  hi16 = ((rg + 1) * nrow16) // (_NC * _NRG)
  nq = hi16 - lo16

  def _zero(i, carry):
    for j in range(_CW // 16):
      acc[i, pl.ds(j * 16, 16)] = jnp.zeros((16,), jnp.float32)
    cacc[i, pl.ds(0, 16)] = jnp.zeros((16,), jnp.float32)
    return carry
  lax.fori_loop(0, _CPAD, _zero, 0)

  @pl.when(wid == 0)
  def _targets():
    for off in (0, 128):
      pltpu.sync_copy(indexes.at[pl.ds(off, 128)], idxbuf)
      pltpu.async_copy(labels.at[idxbuf], tgtbuf, sem).wait()
      pltpu.sync_copy(tgtbuf, tgt_out.at[pl.ds(off, 128)])

  ones16 = jnp.ones((16,), jnp.float32)
  iota16 = lax.broadcasted_iota(jnp.int32, (16,), 0)
  is_cnt = cg == 0
  start = lo16 * 16
  end = hi16 * 16

  def _base(c):
    return jnp.minimum(start + c * _K, end - _K)

  def _fdst(slot):
    return fbuf.at[pl.ds(slot * _K, _K), pl.ds(0, _CW)]

  def _start_dmas(c, slot):
    b = _base(c)
    pltpu.make_async_copy(
        features.at[pl.ds(b, _K), pl.ds(c0, _CW)], _fdst(slot),
        fsem.at[slot]).start()
    pltpu.make_async_copy(labels.at[pl.ds(b, _K)], lbl2.at[slot],
                          lsem.at[slot]).start()

  def _wait_dmas(c, slot):
    b = _base(c)
    pltpu.make_async_copy(
        features.at[pl.ds(b, _K), pl.ds(c0, _CW)], _fdst(slot),
        fsem.at[slot]).wait()
    pltpu.make_async_copy(labels.at[pl.ds(b, _K)], lbl2.at[slot],
                          lsem.at[slot]).wait()

  def _process(c, slot):
    base = _base(c)
    cutoff = start + c * _K

    def _rows16(q, carry):
      lv = lbl2[slot, pl.ds(q * 16, 16)]
      rows = base + q * 16 + iota16
      lbl16 = jnp.where(rows >= cutoff, lv, _DUMP)
      rowg16 = slot * _K + q * 16 + iota16
      for c in range(_CW):
        c16 = (iota16 + c) & (_CW - 1)
        v = plsc.load_gather(fbuf, [rowg16, c16])
        plsc.addupdate_scatter(acc, [lbl16, c16], v)
      @pl.when(is_cnt)
      def _():
        plsc.addupdate_scatter(cacc, [lbl16, iota16], ones16)
      return carry
    lax.fori_loop(0, _K // 16, _rows16, 0)

  _start_dmas(0, 0)
  for c in range(_NCHUNK):
    if c + 1 < _NCHUNK:
      _start_dmas(c + 1, (c + 1) % 2)
    _wait_dmas(c, c % 2)
    _process(c, c % 2)

  pltpu.sync_copy(acc, csum_out.at[cid, sid])
  @pl.when(is_cnt)
  def _dump_cnt():
    pltpu.sync_copy(cacc, cnt_out.at[cid, rl])


@jax.jit
def _sc_stage(features, labels, indexes):
  mesh = plsc.VectorSubcoreMesh(core_axis_name="c", subcore_axis_name="s",
                                num_cores=_NC, num_subcores=_NS)
  return pl.kernel(
      _sc_body,
      out_type=(
          jax.ShapeDtypeStruct((_NC, _NS, _CPAD, _CW), jnp.float32),
          jax.ShapeDtypeStruct((_NC, _NRG, _CPAD, 16), jnp.float32),
          jax.ShapeDtypeStruct((_BATCH,), jnp.int32),
      ),
      mesh=mesh,
      compiler_params=pltpu.CompilerParams(use_tc_tiling_on_sc=False,
                                           needs_layout_passes=False),
      scratch_types=[
          pltpu.VMEM((_CPAD, _CW), jnp.float32),
          pltpu.VMEM((_CPAD, 16), jnp.float32),
          pltpu.VMEM((2 * _K, _CW), jnp.float32),
          pltpu.VMEM((2, _K), jnp.int32),
          pltpu.VMEM((128,), jnp.int32),
          pltpu.VMEM((128,), jnp.int32),
          pltpu.SemaphoreType.DMA((2,)),
          pltpu.SemaphoreType.DMA((2,)),
          pltpu.SemaphoreType.DMA,
      ],
  )(features, labels, indexes)


def _tc_body(inp_ref, csum_ref, cnt_ref, tgt_ref, out_ref):
  blocks = []
  for g in range(_NCG):
    blk = csum_ref[0, g]
    for cid in range(_NC):
      for rl in range(_NRG):
        if cid == 0 and rl == 0:
          continue
        blk = blk + csum_ref[cid, rl * _NCG + g]
    blocks.append(blk)
  csum = jnp.concatenate(blocks, axis=1)
  cnt2 = cnt_ref[0, 0]
  for cid in range(_NC):
    for rl in range(_NRG):
      if cid == 0 and rl == 0:
        continue
      cnt2 = cnt2 + cnt_ref[cid, rl]
  sim = lax.dot_general(inp_ref[...], csum, (((1,), (1,)), ((), ())),
                        preferred_element_type=jnp.float32)
  w = jnp.full((1, 16), 1.0, jnp.float32)
  nums = lax.dot_general(w, cnt2, (((1,), (1,)), ((), ())),
                         preferred_element_type=jnp.float32)
  iota_c = lax.broadcasted_iota(jnp.int32, (1, _CPAD), 1)
  maskf = jnp.where(jnp.logical_and(nums > 0.5, iota_c < _NUM_CLASSES),
                    1.0, 0.0).astype(jnp.float32)
  denom = _TEMP * jnp.where(nums > 0.5, nums, 1.0)
  exps = jnp.exp(sim / denom) * maskf
  sums = jnp.sum(exps, axis=1, keepdims=True) + 1e-6
  logp = jnp.log(exps / sums + 1e-6)
  iota_bc = lax.broadcasted_iota(jnp.int32, (_BATCH, _CPAD), 1)
  onehot = jnp.where(iota_bc == tgt_ref[...], 1.0, 0.0)
  picked = jnp.sum(logp * onehot, axis=1, keepdims=True)
  out_ref[...] = -jnp.sum(picked, axis=0, keepdims=True) / float(_BATCH)


@jax.jit
def _tc_stage(inputs, csum_p, cnt_p, tgt):
  return pl.pallas_call(
      _tc_body,
      out_shape=jax.ShapeDtypeStruct((1, 1), jnp.float32),
  )(inputs, csum_p, cnt_p, tgt)


def kernel(inputs, indexes, features, labels):
  csum_p, cnt_p, tgt = _sc_stage(features, labels, indexes)
  loss = _tc_stage(inputs, csum_p, cnt_p, tgt.reshape(_BATCH, 1))
  return loss[0, 0]
